# baseline (speedup 1.0000x reference)
import functools

import jax
import jax.numpy as jnp
from jax import lax
from jax.experimental import pallas as pl
from jax.experimental.pallas import tpu as pltpu
from jax.experimental.pallas import tpu_sc as plsc

N_NODES = 10000
N_EDGES = 320000
D_FEAT = 128
DH = D_FEAT // 2

NC = 2
NS = 16
C = 80
K = 250
NBUF = 5
ROWS_PT = 632
N_PAD = NS * ROWS_PT
LAST_ROWS = N_NODES - (NS - 1) * ROWS_PT
IDX_MASK = (1 << 14) - 1


def _sc_graph_conv(xlo, xhi, pidx_p, w_p):
    mesh = plsc.VectorSubcoreMesh(core_axis_name="c", subcore_axis_name="s",
                                  num_cores=NC, num_subcores=NS)

    @functools.partial(
        pl.kernel,
        out_type=jax.ShapeDtypeStruct((N_NODES, D_FEAT), jnp.float32),
        mesh=mesh,
        compiler_params=pltpu.CompilerParams(use_tc_tiling_on_sc=False),
        scratch_types=(
            [
                pltpu.VMEM((K, C), jnp.int32),
                pltpu.VMEM((K, C), jnp.float32),
            ]
            + [pltpu.VMEM((C,), jnp.int32) for _ in range(NBUF)]
            + [pltpu.VMEM((C,), jnp.int32) for _ in range(NBUF)]
            + [pltpu.VMEM((C, DH // 2), jnp.int32) for _ in range(NBUF)]
            + [pltpu.VMEM((C, DH), jnp.float32) for _ in range(NBUF)]
            + [pltpu.VMEM_SHARED((N_PAD, DH), jnp.float32)]
            + [pltpu.SemaphoreType.DMA for _ in range(2 * NBUF)]
        ),
    )
    def body(xlo_hbm, xhi_hbm, pidx_hbm, w_hbm, out, pidx_v, w_v, *rest):
        sring = rest[0:NBUF]
        tring = rest[NBUF:2 * NBUF]
        gbufs = rest[2 * NBUF:3 * NBUF]
        fbufs = rest[3 * NBUF:4 * NBUF]
        acc = rest[4 * NBUF]
        gsem = rest[4 * NBUF + 1:4 * NBUF + 1 + NBUF]
        ssem = rest[4 * NBUF + 1 + NBUF:4 * NBUF + 1 + 2 * NBUF]
        cid = lax.axis_index("c")
        sid = lax.axis_index("s")

        pltpu.sync_copy(pidx_hbm.at[sid], pidx_v)
        pltpu.sync_copy(w_hbm.at[sid], w_v)

        zbuf = fbufs[0]

        def zrow(i, carry):
            for f in range(DH // 16):
                zbuf[i, pl.ds(f * 16, 16)] = jnp.zeros((16,), jnp.float32)
            return carry

        lax.fori_loop(0, C, zrow, 0)
        base = sid * ROWS_PT
        nfull = ROWS_PT // C
        rem = ROWS_PT - nfull * C
        for q in range(nfull):
            pltpu.sync_copy(zbuf, acc.at[pl.ds(base + q * C, C)])
        if rem:
            pltpu.sync_copy(zbuf.at[pl.ds(0, rem)],
                            acc.at[pl.ds(base + nfull * C, rem)])
        plsc.subcore_barrier()

        def unpack_idx(jj, b):
            @plsc.parallel_loop(0, C // 16)
            def u(g):
                p = pidx_v[jj, pl.ds(g * 16, 16)]
                sring[b][pl.ds(g * 16, 16)] = p & IDX_MASK
                tring[b][pl.ds(g * 16, 16)] = (p >> 14) & IDX_MASK

        def g_start(jj, b):
            @pl.when(cid == 0)
            def _():
                pltpu.async_copy(xlo_hbm.at[sring[b]], gbufs[b], gsem[b])

            @pl.when(cid == 1)
            def _():
                pltpu.async_copy(xhi_hbm.at[sring[b]], gbufs[b], gsem[b])

        def g_wait(jj, b):
            pltpu.make_async_copy(xlo_hbm.at[sring[b]], gbufs[b],
                                  gsem[b]).wait()

        def s_start(jj, b):
            pltpu.async_copy(fbufs[b], acc.at[tring[b]], ssem[b], add=True)

        def s_wait(jj, b):
            pltpu.make_async_copy(fbufs[b], acc.at[tring[b]], ssem[b]).wait()

        def scale(jj, b):
            src = gbufs[b]
            dst = fbufs[b]

            @plsc.parallel_loop(0, C // 16, unroll=2)
            def grp(g):
                wv = w_v[jj, pl.ds(g * 16, 16)]
                for e in range(16):
                    ws = wv[e]
                    r = g * 16 + e
                    for f in range(DH // 32):
                        pair = src[r, pl.ds(f * 16, 16)]
                        even = lax.bitcast_convert_type(
                            pair << 16, jnp.float32)
                        odd = lax.bitcast_convert_type(
                            pair & jnp.int32(-65536), jnp.float32)
                        dst[r, pl.ds(f * 32, 16)] = even * ws
                        dst[r, pl.ds(f * 32 + 16, 16)] = odd * ws

        for jj in (0, 1):
            unpack_idx(jj, jj)
            g_start(jj, jj)

        def step(i, carry):
            j = i * NBUF
            for b in range(NBUF):
                jj = j + b
                b2 = (b + 2) % NBUF

                @pl.when(jj >= 3)
                def _():
                    s_wait(jj - 3, b2)

                @pl.when(jj + 2 < K)
                def _():
                    unpack_idx(jj + 2, b2)
                    g_start(jj + 2, b2)

                g_wait(jj, b)
                scale(jj, b)
                s_start(jj, b)
            return carry

        lax.fori_loop(0, K // NBUF, step, 0)
        for jj in range(K - 3, K):
            s_wait(jj, jj % NBUF)

        plsc.subcore_barrier()

        col = cid * DH

        @pl.when(sid < NS - 1)
        def _():
            pltpu.sync_copy(acc.at[pl.ds(base, ROWS_PT)],
                            out.at[pl.ds(base, ROWS_PT), pl.ds(col, DH)])

        @pl.when(sid == NS - 1)
        def _():
            pltpu.sync_copy(acc.at[pl.ds(base, LAST_ROWS)],
                            out.at[pl.ds(base, LAST_ROWS), pl.ds(col, DH)])

    return body(xlo, xhi, pidx_p, w_p)


def kernel(input, eidx, enorm, esgn):
    sidx = eidx[0].astype(jnp.int32)
    tidx = eidx[1].astype(jnp.int32)
    pidx_p = (sidx | (tidx << 14)).reshape(NS, K, C)
    w_p = (enorm * esgn).reshape(NS, K, C)
    xbf = input.astype(jnp.bfloat16)
    xperm = xbf.reshape(N_NODES, D_FEAT // 32, 2, 16).swapaxes(2, 3)
    xperm = xperm.reshape(N_NODES, D_FEAT)
    xlo = lax.bitcast_convert_type(
        xperm[:, :DH].reshape(N_NODES, DH // 2, 2), jnp.int32)
    xhi = lax.bitcast_convert_type(
        xperm[:, DH:].reshape(N_NODES, DH // 2, 2), jnp.int32)
    return _sc_graph_conv(xlo, xhi, pidx_p, w_p)


# speedup vs baseline: 1.5140x; 1.5140x over previous
import functools

import jax
import jax.numpy as jnp
from jax import lax
from jax.experimental import pallas as pl
from jax.experimental.pallas import tpu as pltpu
from jax.experimental.pallas import tpu_sc as plsc

N_NODES = 10000
N_EDGES = 320000
D_FEAT = 128
DH = D_FEAT // 2

NC = 2
NS = 16
C = 80
K = 250
EPT = K * C
NBUF = 5
NFH = DH // 16
ROWS_PT = 640
N_PAD = NS * ROWS_PT
LAST_ROWS = N_NODES - (NS - 1) * ROWS_PT


def _sc_graph_conv(xlo, xhi, sidx_p, tidx_p, w_p):
    mesh = plsc.VectorSubcoreMesh(core_axis_name="c", subcore_axis_name="s",
                                  num_cores=NC, num_subcores=NS)

    @functools.partial(
        pl.kernel,
        out_type=jax.ShapeDtypeStruct((N_NODES, D_FEAT), jnp.float32),
        mesh=mesh,
        compiler_params=pltpu.CompilerParams(use_tc_tiling_on_sc=False),
        scratch_types=(
            [
                pltpu.VMEM((K, C), jnp.int32),
                pltpu.VMEM((K, C), jnp.int32),
                pltpu.VMEM((K, C), jnp.float32),
            ]
            + [pltpu.VMEM((C, DH), jnp.float32) for _ in range(NBUF)]
            + [pltpu.VMEM_SHARED((N_PAD, DH), jnp.float32)]
            + [pltpu.SemaphoreType.DMA for _ in range(2 * NBUF)]
        ),
    )
    def body(xlo_hbm, xhi_hbm, sidx_hbm, tidx_hbm, w_hbm, out,
             sidx_v, tidx_v, w_v, b0, b1, b2, b3, b4, acc,
             g0, g1, g2, g3, g4, s0, s1, s2, s3, s4):
        bufs = (b0, b1, b2, b3, b4)
        gsem = (g0, g1, g2, g3, g4)
        ssem = (s0, s1, s2, s3, s4)
        cid = lax.axis_index("c")
        sid = lax.axis_index("s")

        pltpu.sync_copy(sidx_hbm.at[sid], sidx_v)
        pltpu.sync_copy(tidx_hbm.at[sid], tidx_v)
        pltpu.sync_copy(w_hbm.at[sid], w_v)

        zbuf = bufs[0]

        def zrow(i, carry):
            for f in range(NFH):
                zbuf[i, pl.ds(f * 16, 16)] = jnp.zeros((16,), jnp.float32)
            return carry

        lax.fori_loop(0, C, zrow, 0)
        base = sid * ROWS_PT
        nfull = ROWS_PT // C
        rem = ROWS_PT - nfull * C
        for q in range(nfull):
            pltpu.sync_copy(zbuf, acc.at[pl.ds(base + q * C, C)])
        if rem:
            pltpu.sync_copy(zbuf.at[pl.ds(0, rem)],
                            acc.at[pl.ds(base + nfull * C, rem)])
        plsc.subcore_barrier()

        def g_start(jj, b):
            @pl.when(cid == 0)
            def _():
                pltpu.async_copy(xlo_hbm.at[sidx_v.at[jj]], bufs[b], gsem[b])

            @pl.when(cid == 1)
            def _():
                pltpu.async_copy(xhi_hbm.at[sidx_v.at[jj]], bufs[b], gsem[b])

        def g_wait(jj, b):
            pltpu.make_async_copy(xlo_hbm.at[sidx_v.at[jj]], bufs[b],
                                  gsem[b]).wait()

        def s_start(jj, b):
            pltpu.async_copy(bufs[b], acc.at[tidx_v.at[jj]], ssem[b],
                             add=True)

        def s_wait(jj, b):
            pltpu.make_async_copy(bufs[b], acc.at[tidx_v.at[jj]],
                                  ssem[b]).wait()

        def scale(jj, b):
            buf = bufs[b]

            @plsc.parallel_loop(0, C // 16, unroll=2)
            def grp(g):
                wv = w_v[jj, pl.ds(g * 16, 16)]
                for e in range(16):
                    ws = wv[e]
                    r = g * 16 + e
                    for f in range(NFH):
                        buf[r, pl.ds(f * 16, 16)] = (
                            buf[r, pl.ds(f * 16, 16)] * ws)

        g_start(0, 0)
        g_start(1, 1)
        g_start(2, 2)

        def step(i, carry):
            j = i * NBUF
            for b in range(NBUF):
                jj = j + b
                b3 = (b + 3) % NBUF

                @pl.when(jj >= NBUF - 3)
                def _():
                    s_wait(jj - (NBUF - 3), b3)

                @pl.when(jj + 3 < K)
                def _():
                    g_start(jj + 3, b3)

                g_wait(jj, b)
                scale(jj, b)
                s_start(jj, b)
            return carry

        lax.fori_loop(0, K // NBUF, step, 0)
        for jj in range(K - NBUF + 3, K):
            s_wait(jj, jj % NBUF)

        plsc.subcore_barrier()

        col = cid * DH

        @pl.when(sid < NS - 1)
        def _():
            pltpu.sync_copy(acc.at[pl.ds(base, ROWS_PT)],
                            out.at[pl.ds(base, ROWS_PT), pl.ds(col, DH)])

        @pl.when(sid == NS - 1)
        def _():
            pltpu.sync_copy(acc.at[pl.ds(base, LAST_ROWS)],
                            out.at[pl.ds(base, LAST_ROWS), pl.ds(col, DH)])

    return body(xlo, xhi, sidx_p, tidx_p, w_p)


def kernel(input, eidx, enorm, esgn):
    sidx_p = eidx[0].astype(jnp.int32).reshape(NS, K, C)
    tidx_p = eidx[1].astype(jnp.int32).reshape(NS, K, C)
    w_p = (enorm * esgn).reshape(NS, K, C)
    return _sc_graph_conv(input[:, :DH], input[:, DH:],
                          sidx_p, tidx_p, w_p)
